# baseline (speedup 1.0000x reference)
import functools

import jax
import jax.numpy as jnp
from jax import lax
from jax.experimental import pallas as pl
from jax.experimental.pallas import tpu as pltpu
from jax.experimental.pallas import tpu_sc as plsc

N_NODES = 10000
IN_CH = 128
HID_CH = 128
OUT_CH = 64
D = 64

NPAD = 10240
NTILES = 32
CHUNK = 128
NCHUNK = 80
EPT = CHUNK * NCHUNK
EPAD = EPT * NTILES
ROWS_PER_TILE = NPAD // 16

AGG_NCH1 = (160, 0)
AGG_NCH2 = (152, 8)
AGG_MAX = 160


@functools.lru_cache(maxsize=1)
def _sc_mesh():
    return plsc.VectorSubcoreMesh(core_axis_name="c", subcore_axis_name="s")


def _deg_body(dstp_hbm, out_hbm, dst_v, ones_v, zero_v, acc):
    c = lax.axis_index("c")
    s = lax.axis_index("s")
    tid = c * 16 + s
    pltpu.sync_copy(dstp_hbm.at[tid], dst_v)

    one16 = jnp.full((16,), 1.0, dtype=jnp.float32)
    zer16 = jnp.zeros((16,), dtype=jnp.float32)

    def fill(r, _):
        ones_v[r] = one16
        zero_v[r] = zer16
        return ()
    lax.fori_loop(0, CHUNK, fill, ())

    for i in range(ROWS_PER_TILE // CHUNK):
        pltpu.sync_copy(zero_v, acc.at[pl.ds(s * ROWS_PER_TILE + i * CHUNK, CHUNK)])
    plsc.subcore_barrier()

    def body(j, _):
        pltpu.sync_copy(ones_v, acc.at[dst_v.at[j]], add=True)
        return ()
    lax.fori_loop(0, NCHUNK, body, ())
    plsc.subcore_barrier()

    pltpu.sync_copy(acc.at[pl.ds(s * ROWS_PER_TILE, ROWS_PER_TILE)],
                    out_hbm.at[c, pl.ds(s * ROWS_PER_TILE, ROWS_PER_TILE)])


@functools.lru_cache(maxsize=1)
def _deg_kernel():
    return pl.kernel(
        _deg_body,
        out_type=jax.ShapeDtypeStruct((2, NPAD, 16), jnp.float32),
        mesh=_sc_mesh(),
        scratch_types=[
            pltpu.VMEM((NCHUNK, CHUNK), jnp.int32),
            pltpu.VMEM((CHUNK, 16), jnp.float32),
            pltpu.VMEM((CHUNK, 16), jnp.float32),
            pltpu.VMEM_SHARED((NPAD, 16), jnp.float32),
        ],
        compiler_params=pltpu.CompilerParams(use_tc_tiling_on_sc=False),
    )


NBUF = 4


def _make_agg_body(nt, nch_pair):
    def _agg_body(h_hbm, srcp_hbm, dstp_hbm, out_hbm,
                  src_v, dst_v, rows_v, zero_v, acc, *sems):
        c = lax.axis_index("c")
        s = lax.axis_index("s")
        tid = c * 16 + s
        pltpu.sync_copy(srcp_hbm.at[tid], src_v)
        pltpu.sync_copy(dstp_hbm.at[tid], dst_v)

        nch = jnp.where(c == 0, nch_pair[0], nch_pair[1])

        zer16 = jnp.zeros((16,), dtype=jnp.float32)

        def fill(r, _):
            for l in range(D // 16):
                zero_v[r, pl.ds(l * 16, 16)] = zer16
            return ()
        lax.fori_loop(0, CHUNK, fill, ())

        for t in range(nt):
            for i in range(ROWS_PER_TILE // CHUNK):
                pltpu.sync_copy(
                    zero_v, acc.at[pl.ds(s * ROWS_PER_TILE + i * CHUNK, CHUNK)])
            plsc.subcore_barrier()

            ht = h_hbm.at[t]
            for b in range(NBUF):
                @pl.when(b < nch)
                def _():
                    pltpu.make_async_copy(
                        ht.at[src_v.at[b]], rows_v.at[b], sems[b]).start()

            def body(it, _):
                jj = it * NBUF
                for b in range(NBUF):
                    j = jj + b
                    pltpu.make_async_copy(
                        ht.at[src_v.at[j]], rows_v.at[b], sems[b]).wait()
                    pltpu.sync_copy(rows_v.at[b], acc.at[dst_v.at[j]], add=True)

                    @pl.when(j + NBUF < nch)
                    def _():
                        pltpu.make_async_copy(
                            ht.at[src_v.at[j + NBUF]], rows_v.at[b],
                            sems[b]).start()
                return ()
            lax.fori_loop(0, nch // NBUF, body, ())
            plsc.subcore_barrier()

            pltpu.sync_copy(
                acc.at[pl.ds(s * ROWS_PER_TILE, ROWS_PER_TILE)],
                out_hbm.at[t, c, pl.ds(s * ROWS_PER_TILE, ROWS_PER_TILE)])
    return _agg_body


def _make_agg(nt, nch_pair):
    return pl.kernel(
        _make_agg_body(nt, nch_pair),
        out_type=jax.ShapeDtypeStruct((nt, 2, NPAD, D), jnp.float32),
        mesh=_sc_mesh(),
        scratch_types=[
            pltpu.VMEM((AGG_MAX, CHUNK), jnp.int32),
            pltpu.VMEM((AGG_MAX, CHUNK), jnp.int32),
            pltpu.VMEM((NBUF, CHUNK, D), jnp.float32),
            pltpu.VMEM((CHUNK, D), jnp.float32),
            pltpu.VMEM_SHARED((NPAD, D), jnp.float32),
        ] + [pltpu.SemaphoreType.DMA] * NBUF,
        compiler_params=pltpu.CompilerParams(use_tc_tiling_on_sc=False),
    )


@functools.lru_cache(maxsize=None)
def _agg_kernel(nt, nch_pair):
    return _make_agg(nt, nch_pair)


_BR = 512


def _dinv_of(dp_ref):
    return lax.rsqrt(dp_ref[0, :, 0:1] + dp_ref[1, :, 0:1] + 1.0)


def _prep_body(x_ref, w1_ref, dp_ref, o_ref):
    dinv = _dinv_of(dp_ref)
    h = jnp.dot(x_ref[...], w1_ref[...], preferred_element_type=jnp.float32)
    h = h * dinv
    o_ref[0] = h[:, :D]
    o_ref[1] = h[:, D:]


def _mid_body(pa_ref, pb_ref, h_ref, dp_ref, b1_ref, w2_ref, o_ref):
    dinv = _dinv_of(dp_ref)
    left = pa_ref[0] + pa_ref[1] + h_ref[0]
    right = pb_ref[0] + pb_ref[1] + h_ref[1]
    a = jnp.concatenate([left, right], axis=1) * dinv + b1_ref[...]
    a = jnp.maximum(a, 0.0)
    h2 = jnp.dot(a, w2_ref[...], preferred_element_type=jnp.float32)
    o_ref[...] = h2 * dinv


def _fin_body(q_ref, h_ref, dp_ref, b2_ref, o_ref):
    dinv = _dinv_of(dp_ref)
    o_ref[...] = (q_ref[0] + q_ref[1] + h_ref[...]) * dinv + b2_ref[...]


_GRID = (NPAD // _BR,)
_dp_spec = pl.BlockSpec((2, _BR, 16), lambda i: (0, i, 0))


def _row_spec(d):
    return pl.BlockSpec((_BR, d), lambda i: (i, 0))


def _full_spec(a, b):
    return pl.BlockSpec((a, b), lambda i: (0, 0))


def _part_spec(d):
    return pl.BlockSpec((2, _BR, d), lambda i: (0, i, 0))


_prep_call = pl.pallas_call(
    _prep_body, grid=_GRID,
    in_specs=[_row_spec(IN_CH), _full_spec(IN_CH, HID_CH), _dp_spec],
    out_specs=_part_spec(D),
    out_shape=jax.ShapeDtypeStruct((2, NPAD, D), jnp.float32))

_mid_call = pl.pallas_call(
    _mid_body, grid=_GRID,
    in_specs=[_part_spec(D), _part_spec(D), _part_spec(D), _dp_spec,
              _full_spec(1, HID_CH), _full_spec(HID_CH, OUT_CH)],
    out_specs=_row_spec(OUT_CH),
    out_shape=jax.ShapeDtypeStruct((NPAD, OUT_CH), jnp.float32))

_fin_call = pl.pallas_call(
    _fin_body, grid=_GRID,
    in_specs=[_part_spec(OUT_CH), _row_spec(OUT_CH), _dp_spec,
              _full_spec(1, OUT_CH)],
    out_specs=_row_spec(OUT_CH),
    out_shape=jax.ShapeDtypeStruct((NPAD, OUT_CH), jnp.float32))


def kernel(x, edge_index, W1, b1, W2, b2):
    src = edge_index[0].astype(jnp.int32)
    dst = edge_index[1].astype(jnp.int32)
    n_extra = EPAD - src.shape[0]
    src_flat = jnp.pad(src, (0, n_extra))
    pad_dst = N_NODES + (jnp.arange(n_extra, dtype=jnp.int32) % (NPAD - N_NODES))
    dst_flat = jnp.concatenate([dst, pad_dst])
    dstp = dst_flat.reshape(NTILES, NCHUNK, CHUNK)

    def _split(flat, nch):
        e0 = 16 * nch[0] * CHUNK
        c0 = flat[:e0].reshape(16, nch[0], CHUNK)
        c1 = flat[e0:].reshape(16, nch[1], CHUNK)
        c1 = jnp.pad(c1, ((0, 0), (0, AGG_MAX - nch[1]), (0, 0)))
        c0 = jnp.pad(c0, ((0, 0), (0, AGG_MAX - nch[0]), (0, 0)))
        return jnp.concatenate([c0, c1], axis=0)

    srcp_1 = _split(src_flat, AGG_NCH1)
    dstp_1 = _split(dst_flat, AGG_NCH1)
    srcp_2 = _split(src_flat, AGG_NCH2)
    dstp_2 = _split(dst_flat, AGG_NCH2)
    x_pad = jnp.pad(x, ((0, NPAD - N_NODES), (0, 0)))

    degp = _deg_kernel()(dstp)
    h1s = _prep_call(x_pad, W1, degp)
    p = _agg_kernel(2, AGG_NCH1)(h1s, srcp_1, dstp_1)
    h2s = _mid_call(p[0], p[1], h1s, degp, b1.reshape(1, HID_CH), W2)
    q = _agg_kernel(1, AGG_NCH2)(h2s[None], srcp_2, dstp_2)[0]
    out = _fin_call(q, h2s, degp, b2.reshape(1, OUT_CH))
    return out[:N_NODES]


# speedup vs baseline: 1.9953x; 1.9953x over previous
import functools

import jax
import jax.numpy as jnp
from jax import lax
from jax.experimental import pallas as pl
from jax.experimental.pallas import tpu as pltpu
from jax.experimental.pallas import tpu_sc as plsc

N_NODES = 10000
IN_CH = 128
HID_CH = 128
OUT_CH = 64
D = 64

NPAD = 10240
NTILES = 32
CHUNK = 128
NCHUNK = 80
EPT = CHUNK * NCHUNK
EPAD = EPT * NTILES
ROWS_PER_TILE = NPAD // 16

AGG_NCH1 = (160, 0)
AGG_NCH2 = (152, 8)
AGG_MAX = 160


@functools.lru_cache(maxsize=1)
def _sc_mesh():
    return plsc.VectorSubcoreMesh(core_axis_name="c", subcore_axis_name="s")


def _deg_body(dstp_hbm, out_hbm, dst_v, ones_v, zero_v, acc):
    c = lax.axis_index("c")
    s = lax.axis_index("s")
    tid = c * 16 + s
    pltpu.sync_copy(dstp_hbm.at[tid], dst_v)

    one16 = jnp.full((16,), 1.0, dtype=jnp.float32)
    zer16 = jnp.zeros((16,), dtype=jnp.float32)

    def fill(r, _):
        ones_v[r] = one16
        zero_v[r] = zer16
        return ()
    lax.fori_loop(0, CHUNK, fill, ())

    for i in range(ROWS_PER_TILE // CHUNK):
        pltpu.sync_copy(zero_v, acc.at[pl.ds(s * ROWS_PER_TILE + i * CHUNK, CHUNK)])
    plsc.subcore_barrier()

    def body(j, _):
        pltpu.sync_copy(ones_v, acc.at[dst_v.at[j]], add=True)
        return ()
    lax.fori_loop(0, NCHUNK, body, ())
    plsc.subcore_barrier()

    pltpu.sync_copy(acc.at[pl.ds(s * ROWS_PER_TILE, ROWS_PER_TILE)],
                    out_hbm.at[c, pl.ds(s * ROWS_PER_TILE, ROWS_PER_TILE)])


@functools.lru_cache(maxsize=1)
def _deg_kernel():
    return pl.kernel(
        _deg_body,
        out_type=jax.ShapeDtypeStruct((2, NPAD, 16), jnp.float32),
        mesh=_sc_mesh(),
        scratch_types=[
            pltpu.VMEM((NCHUNK, CHUNK), jnp.int32),
            pltpu.VMEM((CHUNK, 16), jnp.float32),
            pltpu.VMEM((CHUNK, 16), jnp.float32),
            pltpu.VMEM_SHARED((NPAD, 16), jnp.float32),
        ],
        compiler_params=pltpu.CompilerParams(use_tc_tiling_on_sc=False),
    )


NBUF = 4


def _make_agg_body(nt, nch_pair):
    def _agg_body(h_hbm, srcp_hbm, dstp_hbm, out_hbm,
                  src_v, dst_v, rows_v, zero_v, acc, *sems):
        c = lax.axis_index("c")
        s = lax.axis_index("s")
        tid = c * 16 + s
        pltpu.sync_copy(srcp_hbm.at[tid], src_v)
        pltpu.sync_copy(dstp_hbm.at[tid], dst_v)

        nch = jnp.where(c == 0, nch_pair[0], nch_pair[1])

        zer16 = jnp.zeros((16,), dtype=jnp.float32)

        def fill(r, _):
            for l in range(D // 16):
                zero_v[r, pl.ds(l * 16, 16)] = zer16
            return ()
        lax.fori_loop(0, CHUNK, fill, ())

        for t in range(nt):
            for i in range(ROWS_PER_TILE // CHUNK):
                pltpu.sync_copy(
                    zero_v, acc.at[pl.ds(s * ROWS_PER_TILE + i * CHUNK, CHUNK)])
            plsc.subcore_barrier()

            ht = h_hbm.at[t]
            for b in range(NBUF):
                @pl.when(b < nch)
                def _():
                    pltpu.make_async_copy(
                        ht.at[src_v.at[b]], rows_v.at[b], sems[b]).start()

            def body(it, _):
                jj = it * NBUF
                for b in range(NBUF):
                    j = jj + b
                    pltpu.make_async_copy(
                        ht.at[src_v.at[j]], rows_v.at[b], sems[b]).wait()
                    pltpu.sync_copy(rows_v.at[b], acc.at[dst_v.at[j]], add=True)

                    @pl.when(j + NBUF < nch)
                    def _():
                        pltpu.make_async_copy(
                            ht.at[src_v.at[j + NBUF]], rows_v.at[b],
                            sems[b]).start()
                return ()
            lax.fori_loop(0, nch // NBUF, body, ())
            plsc.subcore_barrier()

            pltpu.sync_copy(
                acc.at[pl.ds(s * ROWS_PER_TILE, ROWS_PER_TILE)],
                out_hbm.at[t, c, pl.ds(s * ROWS_PER_TILE, ROWS_PER_TILE)])
    return _agg_body


def _make_agg(nt, nch_pair):
    return pl.kernel(
        _make_agg_body(nt, nch_pair),
        out_type=jax.ShapeDtypeStruct((nt, 2, NPAD, D), jnp.float32),
        mesh=_sc_mesh(),
        scratch_types=[
            pltpu.VMEM((AGG_MAX, CHUNK), jnp.int32),
            pltpu.VMEM((AGG_MAX, CHUNK), jnp.int32),
            pltpu.VMEM((NBUF, CHUNK, D), jnp.float32),
            pltpu.VMEM((CHUNK, D), jnp.float32),
            pltpu.VMEM_SHARED((NPAD, D), jnp.float32),
        ] + [pltpu.SemaphoreType.DMA] * NBUF,
        compiler_params=pltpu.CompilerParams(use_tc_tiling_on_sc=False),
    )


@functools.lru_cache(maxsize=None)
def _agg_kernel(nt, nch_pair):
    return _make_agg(nt, nch_pair)


_BR = 512


def _dinv_of(dp_ref):
    return lax.rsqrt(dp_ref[0, :, 0:1] + dp_ref[1, :, 0:1] + 1.0)


def _prep_body(x_ref, w1_ref, dp_ref, o_ref):
    dinv = _dinv_of(dp_ref)
    h = jnp.dot(x_ref[...], w1_ref[...], preferred_element_type=jnp.float32)
    h = h * dinv
    o_ref[0] = h[:, :D]
    o_ref[1] = h[:, D:]


def _mid_body(pa_ref, pb_ref, h_ref, dp_ref, b1_ref, w2_ref, o_ref):
    dinv = _dinv_of(dp_ref)
    left = pa_ref[0] + pa_ref[1] + h_ref[0]
    right = pb_ref[0] + pb_ref[1] + h_ref[1]
    a = jnp.concatenate([left, right], axis=1) * dinv + b1_ref[...]
    a = jnp.maximum(a, 0.0)
    h2 = jnp.dot(a, w2_ref[...], preferred_element_type=jnp.float32)
    o_ref[...] = h2 * dinv


def _fin_body(q_ref, h_ref, dp_ref, b2_ref, o_ref):
    dinv = _dinv_of(dp_ref)
    o_ref[...] = (q_ref[0] + q_ref[1] + h_ref[...]) * dinv + b2_ref[...]


_GRID = (NPAD // _BR,)
_dp_spec = pl.BlockSpec((2, _BR, 16), lambda i: (0, i, 0))


def _row_spec(d):
    return pl.BlockSpec((_BR, d), lambda i: (i, 0))


def _full_spec(a, b):
    return pl.BlockSpec((a, b), lambda i: (0, 0))


def _part_spec(d):
    return pl.BlockSpec((2, _BR, d), lambda i: (0, i, 0))


_prep_call = pl.pallas_call(
    _prep_body, grid=_GRID,
    in_specs=[_row_spec(IN_CH), _full_spec(IN_CH, HID_CH), _dp_spec],
    out_specs=_part_spec(D),
    out_shape=jax.ShapeDtypeStruct((2, NPAD, D), jnp.float32))

_mid_call = pl.pallas_call(
    _mid_body, grid=_GRID,
    in_specs=[_part_spec(D), _part_spec(D), _part_spec(D), _dp_spec,
              _full_spec(1, HID_CH), _full_spec(HID_CH, OUT_CH)],
    out_specs=_row_spec(OUT_CH),
    out_shape=jax.ShapeDtypeStruct((NPAD, OUT_CH), jnp.float32))

_fin_call = pl.pallas_call(
    _fin_body, grid=_GRID,
    in_specs=[_part_spec(OUT_CH), _row_spec(OUT_CH), _dp_spec,
              _full_spec(1, OUT_CH)],
    out_specs=_row_spec(OUT_CH),
    out_shape=jax.ShapeDtypeStruct((NPAD, OUT_CH), jnp.float32))


def kernel(x, edge_index, W1, b1, W2, b2):
    src = edge_index[0].astype(jnp.int32)
    dst = edge_index[1].astype(jnp.int32)
    n_extra = EPAD - src.shape[0]
    pad_src = (jnp.arange(n_extra, dtype=jnp.int32) * 1031) % N_NODES
    src_flat = jnp.concatenate([src, pad_src])
    pad_dst = N_NODES + (jnp.arange(n_extra, dtype=jnp.int32) % (NPAD - N_NODES))
    dst_flat = jnp.concatenate([dst, pad_dst])
    dstp = dst_flat.reshape(NTILES, NCHUNK, CHUNK)

    def _split(flat, nch):
        e0 = 16 * nch[0] * CHUNK
        c0 = flat[:e0].reshape(16, nch[0], CHUNK)
        c1 = flat[e0:].reshape(16, nch[1], CHUNK)
        c1 = jnp.pad(c1, ((0, 0), (0, AGG_MAX - nch[1]), (0, 0)))
        c0 = jnp.pad(c0, ((0, 0), (0, AGG_MAX - nch[0]), (0, 0)))
        return jnp.concatenate([c0, c1], axis=0)

    srcp_1 = _split(src_flat, AGG_NCH1)
    dstp_1 = _split(dst_flat, AGG_NCH1)
    srcp_2 = _split(src_flat, AGG_NCH2)
    dstp_2 = _split(dst_flat, AGG_NCH2)
    x_pad = jnp.pad(x, ((0, NPAD - N_NODES), (0, 0)))

    degp = _deg_kernel()(dstp)
    h1s = _prep_call(x_pad, W1, degp)
    p = _agg_kernel(2, AGG_NCH1)(h1s, srcp_1, dstp_1)
    h2s = _mid_call(p[0], p[1], h1s, degp, b1.reshape(1, HID_CH), W2)
    q = _agg_kernel(1, AGG_NCH2)(h2s[None], srcp_2, dstp_2)[0]
    out = _fin_call(q, h2s, degp, b2.reshape(1, OUT_CH))
    return out[:N_NODES]
